# baseline (speedup 1.0000x reference)
import functools

import jax
import jax.numpy as jnp
from jax import lax
from jax.experimental import pallas as pl
from jax.experimental.pallas import tpu as pltpu
from jax.experimental.pallas import tpu_sc as plsc

N_NODES = 10000
N_EDGES = 320000
IN_CH = 128
HID_CH = 256
OUT_CH = 2

W80 = 80
CNT_COL = 48
P16 = 16
NC = 2
NS = 16
N_PAD = 10240
SROWS = N_PAD // NS
KA = 128
TA = 158

KC = 80
EPW_C = N_EDGES // (NC * NS)
TC_CH = 126


def _zero_fill(buf, rows, cols):
    def zrow(r, _):
        def zcol(c, _):
            buf[r, pl.ds(c * 16, 16)] = jnp.zeros((16,), jnp.float32)
            return 0
        return lax.fori_loop(0, cols // 16, zcol, 0)
    lax.fori_loop(0, rows, zrow, 0)


def _zero_shared_slice(buf, shared, base, bufrows):
    for t in range(SROWS // bufrows):
        pltpu.sync_copy(buf, shared.at[pl.ds(base + t * bufrows, bufrows)])


def _edge_loop(table_hbm, src_v, dst_v, shared, b0, b1, g0, g1, nch):
    pltpu.async_copy(table_hbm.at[src_v.at[0]], b0, g0)
    pltpu.async_copy(table_hbm.at[src_v.at[1]], b1, g1)

    def step(j, buf, sem):
        pltpu.make_async_copy(table_hbm.at[src_v.at[j]], buf, sem).wait()
        pltpu.sync_copy(buf, shared.at[dst_v.at[j]], add=True)
        pltpu.async_copy(table_hbm.at[src_v.at[j + 2]], buf, sem)

    def body(i, _):
        step(2 * i, b0, g0)
        step(2 * i + 1, b1, g1)
        return 0

    lax.fori_loop(0, nch // 2, body, 0)
    pltpu.make_async_copy(table_hbm.at[src_v.at[0]], b0, g0).wait()
    pltpu.make_async_copy(table_hbm.at[src_v.at[0]], b1, g1).wait()


def _sc_agg_wide(x2, src3, dst3):
    mesh = plsc.VectorSubcoreMesh(core_axis_name="c", subcore_axis_name="s")

    @functools.partial(
        pl.kernel,
        out_type=jax.ShapeDtypeStruct((NC, N_PAD, W80), jnp.float32),
        mesh=mesh,
        scratch_types=[
            pltpu.VMEM((TA + 2, KA), jnp.int32),
            pltpu.VMEM((TA, KA), jnp.int32),
            pltpu.VMEM((KA, W80), jnp.float32),
            pltpu.VMEM((KA, W80), jnp.float32),
            pltpu.VMEM_SHARED((N_PAD, W80), jnp.float32),
            pltpu.SemaphoreType.DMA,
            pltpu.SemaphoreType.DMA,
        ],
        compiler_params=pltpu.CompilerParams(use_tc_tiling_on_sc=False),
    )
    def k(x_hbm, src_hbm, dst_hbm, out_hbm, src_v, dst_v, b0, b1,
          shared, g0, g1):
        cid = lax.axis_index("c")
        sid = lax.axis_index("s")
        base = pl.multiple_of(sid * SROWS, SROWS)

        _zero_fill(b0, KA, W80)
        _zero_shared_slice(b0, shared, base, KA)
        pltpu.sync_copy(src_hbm.at[sid], src_v)
        pltpu.sync_copy(dst_hbm.at[sid], dst_v)

        off = jnp.full((16,), cid * N_PAD, jnp.int32)

        def orow(r, _):
            def ocol(c, _):
                src_v[r, pl.ds(c * 16, 16)] = (
                    src_v[r, pl.ds(c * 16, 16)] + off)
                return 0
            return lax.fori_loop(0, KA // 16, ocol, 0)
        lax.fori_loop(0, TA + 2, orow, 0)

        plsc.subcore_barrier()
        _edge_loop(x_hbm, src_v, dst_v, shared, b0, b1, g0, g1, TA)
        plsc.subcore_barrier()

        pltpu.sync_copy(shared.at[pl.ds(base, SROWS)],
                        out_hbm.at[cid, pl.ds(base, SROWS)])

    return k(x2, src3, dst3)


def _sc_agg_small(p_pad, src3, dst3):
    mesh = plsc.VectorSubcoreMesh(core_axis_name="c", subcore_axis_name="s")

    @functools.partial(
        pl.kernel,
        out_type=jax.ShapeDtypeStruct((NC, N_PAD, P16), jnp.float32),
        mesh=mesh,
        scratch_types=[
            pltpu.VMEM((TC_CH + 2, KC), jnp.int32),
            pltpu.VMEM((TC_CH, KC), jnp.int32),
            pltpu.VMEM((KC, P16), jnp.float32),
            pltpu.VMEM((KC, P16), jnp.float32),
            pltpu.VMEM_SHARED((N_PAD, P16), jnp.float32),
            pltpu.SemaphoreType.DMA,
            pltpu.SemaphoreType.DMA,
        ],
        compiler_params=pltpu.CompilerParams(use_tc_tiling_on_sc=False),
    )
    def k(p_hbm, src_hbm, dst_hbm, out_hbm, src_v, dst_v, b0, b1,
          shared, g0, g1):
        cid = lax.axis_index("c")
        sid = lax.axis_index("s")
        wid = cid * NS + sid
        base = pl.multiple_of(sid * SROWS, SROWS)

        _zero_fill(b0, KC, P16)
        _zero_shared_slice(b0, shared, base, KC)
        pltpu.sync_copy(src_hbm.at[wid], src_v)
        pltpu.sync_copy(dst_hbm.at[wid], dst_v)
        plsc.subcore_barrier()
        _edge_loop(p_hbm, src_v, dst_v, shared, b0, b1, g0, g1, TC_CH)
        plsc.subcore_barrier()

        pltpu.sync_copy(shared.at[pl.ds(base, SROWS)],
                        out_hbm.at[cid, pl.ds(base, SROWS)])

    return k(p_pad, src3, dst3)


def _tc_dense(agg2, x2, w1lT, w1rT, b1r, w2lT, w2rT, b2r):
    NB = 1024
    grid = (N_PAD // NB,)

    def body(a_ref, xlo_ref, xhi_ref, w1l_ref, w1r_ref, b1_ref, w2l_ref,
             w2r_ref, b2_ref, pout_ref, qout_ref, inv_ref):
        alo = a_ref[0]
        ahi = a_ref[1]
        cnt = ahi[:, CNT_COL:CNT_COL + 1]
        inv = 1.0 / jnp.maximum(cnt, 1.0)
        mean = jnp.concatenate([alo, ahi[:, :IN_CH - W80]], axis=1) * inv
        x_blk = jnp.concatenate(
            [xlo_ref[...], xhi_ref[...][:, :IN_CH - W80]], axis=1)
        h = (jnp.dot(mean, w1l_ref[...], preferred_element_type=jnp.float32)
             + b1_ref[...]
             + jnp.dot(x_blk, w1r_ref[...],
                       preferred_element_type=jnp.float32))
        h = jnp.maximum(h, 0.0)
        pout_ref[...] = jnp.dot(h, w2l_ref[...],
                                preferred_element_type=jnp.float32)
        qout_ref[...] = (jnp.dot(h, w2r_ref[...],
                                 preferred_element_type=jnp.float32)
                         + b2_ref[...])
        inv_ref[...] = jnp.broadcast_to(inv, (NB, P16))

    return pl.pallas_call(
        body,
        grid=grid,
        in_specs=[
            pl.BlockSpec((NC, NB, W80), lambda i: (0, i, 0)),
            pl.BlockSpec((NB, W80), lambda i: (i, 0)),
            pl.BlockSpec((NB, W80), lambda i: (i + N_PAD // NB, 0)),
            pl.BlockSpec((IN_CH, HID_CH), lambda i: (0, 0)),
            pl.BlockSpec((IN_CH, HID_CH), lambda i: (0, 0)),
            pl.BlockSpec((1, HID_CH), lambda i: (0, 0)),
            pl.BlockSpec((HID_CH, P16), lambda i: (0, 0)),
            pl.BlockSpec((HID_CH, P16), lambda i: (0, 0)),
            pl.BlockSpec((1, P16), lambda i: (0, 0)),
        ],
        out_specs=[
            pl.BlockSpec((NB, P16), lambda i: (i, 0)),
            pl.BlockSpec((NB, P16), lambda i: (i, 0)),
            pl.BlockSpec((NB, P16), lambda i: (i, 0)),
        ],
        out_shape=[
            jax.ShapeDtypeStruct((N_PAD, P16), jnp.float32),
            jax.ShapeDtypeStruct((N_PAD, P16), jnp.float32),
            jax.ShapeDtypeStruct((N_PAD, P16), jnp.float32),
        ],
    )(agg2, x2, x2, w1lT, w1rT, b1r, w2lT, w2rT, b2r)


def _tc_combine(pparts, invb, qb):
    def body(p_ref, i_ref, q_ref, o_ref):
        o_ref[...] = ((p_ref[0] + p_ref[1]) * i_ref[...]) + q_ref[...]

    return pl.pallas_call(
        body,
        out_shape=jax.ShapeDtypeStruct((N_PAD, P16), jnp.float32),
    )(pparts, invb, qb)


def kernel(x, edge_index, W1_l, b1, W1_r, W2_l, b2, W2_r):
    ei = edge_index.astype(jnp.int32)
    src, dst = ei[0], ei[1]

    x_lo = jnp.pad(x[:, :W80], ((0, N_PAD - N_NODES), (0, 0)))
    x_hi = jnp.concatenate(
        [x[:, W80:], jnp.ones((N_NODES, 1), x.dtype),
         jnp.zeros((N_NODES, W80 - CNT_COL - 1), x.dtype)], axis=1)
    x_hi = jnp.pad(x_hi, ((0, N_PAD - N_NODES), (0, 0)))
    x2 = jnp.concatenate([x_lo, x_hi], axis=0)

    def pack(nw, t, k):
        epw = N_EDGES // nw
        pad_e = t * k - epw
        s3 = jnp.pad(src.reshape(nw, epw), ((0, 0), (0, pad_e)))
        s3 = jnp.pad(s3.reshape(nw, t, k), ((0, 0), (0, 2), (0, 0)))
        dummy = jnp.broadcast_to(
            (N_NODES + 100 + jnp.arange(nw, dtype=jnp.int32))[:, None],
            (nw, pad_e))
        d3 = jnp.concatenate(
            [dst.reshape(nw, epw), dummy], axis=1).reshape(nw, t, k)
        return s3, d3

    srcA, dstA = pack(NS, TA, KA)
    srcC, dstC = pack(NC * NS, TC_CH, KC)

    w1lT = W1_l.T
    w1rT = W1_r.T
    w2lT = jnp.pad(W2_l, ((0, P16 - OUT_CH), (0, 0))).T
    w2rT = jnp.pad(W2_r, ((0, P16 - OUT_CH), (0, 0))).T
    b2r = jnp.pad(b2, (0, P16 - OUT_CH)).reshape(1, P16)
    b1r = b1.reshape(1, HID_CH)

    agg2 = _sc_agg_wide(x2, srcA, dstA)
    p_pad, qb, invb = _tc_dense(agg2, x2, w1lT, w1rT, b1r, w2lT, w2rT, b2r)
    pparts = _sc_agg_small(p_pad, srcC, dstC)
    out16 = _tc_combine(pparts, invb, qb)
    return out16[:N_NODES, :OUT_CH]


# speedup vs baseline: 1.0984x; 1.0984x over previous
import functools

import jax
import jax.numpy as jnp
from jax import lax
from jax.experimental import pallas as pl
from jax.experimental.pallas import tpu as pltpu
from jax.experimental.pallas import tpu_sc as plsc

N_NODES = 10000
N_EDGES = 320000
IN_CH = 128
HID_CH = 256
OUT_CH = 2

W80 = 80
CNT_COL = 48
P16 = 16
NC = 2
NS = 16
N_PAD = 10240
SROWS = N_PAD // NS
KA = 64
TA = 314

KC = 80
EPW_C = N_EDGES // (NC * NS)
TC_CH = 126


def _zero_fill(buf, rows, cols):
    def zrow(r, _):
        def zcol(c, _):
            buf[r, pl.ds(c * 16, 16)] = jnp.zeros((16,), jnp.float32)
            return 0
        return lax.fori_loop(0, cols // 16, zcol, 0)
    lax.fori_loop(0, rows, zrow, 0)


def _zero_shared_slice(buf, shared, base, bufrows):
    for t in range(SROWS // bufrows):
        pltpu.sync_copy(buf, shared.at[pl.ds(base + t * bufrows, bufrows)])


def _edge_loop(table_hbm, src_v, dst_v, shared, b0, b1, g0, g1, nch):
    pltpu.async_copy(table_hbm.at[src_v.at[0]], b0, g0)
    pltpu.async_copy(table_hbm.at[src_v.at[1]], b1, g1)

    def step(j, buf, sem):
        pltpu.make_async_copy(table_hbm.at[src_v.at[j]], buf, sem).wait()
        pltpu.sync_copy(buf, shared.at[dst_v.at[j]], add=True)
        pltpu.async_copy(table_hbm.at[src_v.at[j + 2]], buf, sem)

    def body(i, _):
        step(2 * i, b0, g0)
        step(2 * i + 1, b1, g1)
        return 0

    lax.fori_loop(0, nch // 2, body, 0)
    pltpu.make_async_copy(table_hbm.at[src_v.at[0]], b0, g0).wait()
    pltpu.make_async_copy(table_hbm.at[src_v.at[0]], b1, g1).wait()


def _sc_agg_wide(x2, src3, dst3):
    mesh = plsc.VectorSubcoreMesh(core_axis_name="c", subcore_axis_name="s")

    @functools.partial(
        pl.kernel,
        out_type=jax.ShapeDtypeStruct((NC, N_PAD, W80), jnp.float32),
        mesh=mesh,
        scratch_types=[
            pltpu.VMEM((TA + 2, KA), jnp.int32),
            pltpu.VMEM((TA, KA), jnp.int32),
            pltpu.VMEM((KA, W80), jnp.float32),
            pltpu.VMEM((KA, W80), jnp.float32),
            pltpu.VMEM_SHARED((N_PAD, W80), jnp.float32),
            pltpu.SemaphoreType.DMA,
            pltpu.SemaphoreType.DMA,
        ],
        compiler_params=pltpu.CompilerParams(use_tc_tiling_on_sc=False),
    )
    def k(x_hbm, src_hbm, dst_hbm, out_hbm, src_v, dst_v, b0, b1,
          shared, g0, g1):
        cid = lax.axis_index("c")
        sid = lax.axis_index("s")
        base = pl.multiple_of(sid * SROWS, SROWS)

        _zero_fill(b0, KA, W80)
        _zero_shared_slice(b0, shared, base, KA)
        pltpu.sync_copy(src_hbm.at[sid], src_v)
        pltpu.sync_copy(dst_hbm.at[sid], dst_v)

        off = jnp.full((16,), cid * N_PAD, jnp.int32)

        def orow(r, _):
            def ocol(c, _):
                src_v[r, pl.ds(c * 16, 16)] = (
                    src_v[r, pl.ds(c * 16, 16)] + off)
                return 0
            return lax.fori_loop(0, KA // 16, ocol, 0)
        lax.fori_loop(0, TA + 2, orow, 0)

        plsc.subcore_barrier()
        _edge_loop(x_hbm, src_v, dst_v, shared, b0, b1, g0, g1, TA)
        plsc.subcore_barrier()

        pltpu.sync_copy(shared.at[pl.ds(base, SROWS)],
                        out_hbm.at[cid, pl.ds(base, SROWS)])

    return k(x2, src3, dst3)


def _sc_agg_small(p_pad, src3, dst3):
    mesh = plsc.VectorSubcoreMesh(core_axis_name="c", subcore_axis_name="s")

    @functools.partial(
        pl.kernel,
        out_type=jax.ShapeDtypeStruct((NC, N_PAD, P16), jnp.float32),
        mesh=mesh,
        scratch_types=[
            pltpu.VMEM((TC_CH + 2, KC), jnp.int32),
            pltpu.VMEM((TC_CH, KC), jnp.int32),
            pltpu.VMEM((KC, P16), jnp.float32),
            pltpu.VMEM((KC, P16), jnp.float32),
            pltpu.VMEM_SHARED((N_PAD, P16), jnp.float32),
            pltpu.SemaphoreType.DMA,
            pltpu.SemaphoreType.DMA,
        ],
        compiler_params=pltpu.CompilerParams(use_tc_tiling_on_sc=False),
    )
    def k(p_hbm, src_hbm, dst_hbm, out_hbm, src_v, dst_v, b0, b1,
          shared, g0, g1):
        cid = lax.axis_index("c")
        sid = lax.axis_index("s")
        wid = cid * NS + sid
        base = pl.multiple_of(sid * SROWS, SROWS)

        _zero_fill(b0, KC, P16)
        _zero_shared_slice(b0, shared, base, KC)
        pltpu.sync_copy(src_hbm.at[wid], src_v)
        pltpu.sync_copy(dst_hbm.at[wid], dst_v)
        plsc.subcore_barrier()
        _edge_loop(p_hbm, src_v, dst_v, shared, b0, b1, g0, g1, TC_CH)
        plsc.subcore_barrier()

        pltpu.sync_copy(shared.at[pl.ds(base, SROWS)],
                        out_hbm.at[cid, pl.ds(base, SROWS)])

    return k(p_pad, src3, dst3)


def _tc_dense(agg2, x2, w1lT, w1rT, b1r, w2lT, w2rT, b2r):
    NB = 1024
    grid = (N_PAD // NB,)

    def body(a_ref, xlo_ref, xhi_ref, w1l_ref, w1r_ref, b1_ref, w2l_ref,
             w2r_ref, b2_ref, pout_ref, qout_ref, inv_ref):
        alo = a_ref[0]
        ahi = a_ref[1]
        cnt = ahi[:, CNT_COL:CNT_COL + 1]
        inv = 1.0 / jnp.maximum(cnt, 1.0)
        mean = jnp.concatenate([alo, ahi[:, :IN_CH - W80]], axis=1) * inv
        x_blk = jnp.concatenate(
            [xlo_ref[...], xhi_ref[...][:, :IN_CH - W80]], axis=1)
        h = (jnp.dot(mean, w1l_ref[...], preferred_element_type=jnp.float32)
             + b1_ref[...]
             + jnp.dot(x_blk, w1r_ref[...],
                       preferred_element_type=jnp.float32))
        h = jnp.maximum(h, 0.0)
        pout_ref[...] = jnp.dot(h, w2l_ref[...],
                                preferred_element_type=jnp.float32)
        qout_ref[...] = (jnp.dot(h, w2r_ref[...],
                                 preferred_element_type=jnp.float32)
                         + b2_ref[...])
        inv_ref[...] = jnp.broadcast_to(inv, (NB, P16))

    return pl.pallas_call(
        body,
        grid=grid,
        in_specs=[
            pl.BlockSpec((NC, NB, W80), lambda i: (0, i, 0)),
            pl.BlockSpec((NB, W80), lambda i: (i, 0)),
            pl.BlockSpec((NB, W80), lambda i: (i + N_PAD // NB, 0)),
            pl.BlockSpec((IN_CH, HID_CH), lambda i: (0, 0)),
            pl.BlockSpec((IN_CH, HID_CH), lambda i: (0, 0)),
            pl.BlockSpec((1, HID_CH), lambda i: (0, 0)),
            pl.BlockSpec((HID_CH, P16), lambda i: (0, 0)),
            pl.BlockSpec((HID_CH, P16), lambda i: (0, 0)),
            pl.BlockSpec((1, P16), lambda i: (0, 0)),
        ],
        out_specs=[
            pl.BlockSpec((NB, P16), lambda i: (i, 0)),
            pl.BlockSpec((NB, P16), lambda i: (i, 0)),
            pl.BlockSpec((NB, P16), lambda i: (i, 0)),
        ],
        out_shape=[
            jax.ShapeDtypeStruct((N_PAD, P16), jnp.float32),
            jax.ShapeDtypeStruct((N_PAD, P16), jnp.float32),
            jax.ShapeDtypeStruct((N_PAD, P16), jnp.float32),
        ],
    )(agg2, x2, x2, w1lT, w1rT, b1r, w2lT, w2rT, b2r)


def _tc_combine(pparts, invb, qb):
    def body(p_ref, i_ref, q_ref, o_ref):
        o_ref[...] = ((p_ref[0] + p_ref[1]) * i_ref[...]) + q_ref[...]

    return pl.pallas_call(
        body,
        out_shape=jax.ShapeDtypeStruct((N_PAD, P16), jnp.float32),
    )(pparts, invb, qb)


def kernel(x, edge_index, W1_l, b1, W1_r, W2_l, b2, W2_r):
    ei = edge_index.astype(jnp.int32)
    src, dst = ei[0], ei[1]

    x_lo = jnp.pad(x[:, :W80], ((0, N_PAD - N_NODES), (0, 0)))
    x_hi = jnp.concatenate(
        [x[:, W80:], jnp.ones((N_NODES, 1), x.dtype),
         jnp.zeros((N_NODES, W80 - CNT_COL - 1), x.dtype)], axis=1)
    x_hi = jnp.pad(x_hi, ((0, N_PAD - N_NODES), (0, 0)))
    x2 = jnp.concatenate([x_lo, x_hi], axis=0)

    def pack(nw, t, k):
        epw = N_EDGES // nw
        pad_e = t * k - epw
        s3 = jnp.pad(src.reshape(nw, epw), ((0, 0), (0, pad_e)))
        s3 = jnp.pad(s3.reshape(nw, t, k), ((0, 0), (0, 2), (0, 0)))
        dummy = jnp.broadcast_to(
            (N_NODES + 100 + jnp.arange(nw, dtype=jnp.int32))[:, None],
            (nw, pad_e))
        d3 = jnp.concatenate(
            [dst.reshape(nw, epw), dummy], axis=1).reshape(nw, t, k)
        return s3, d3

    srcA, dstA = pack(NS, TA, KA)
    srcC, dstC = pack(NC * NS, TC_CH, KC)

    w1lT = W1_l.T
    w1rT = W1_r.T
    w2lT = jnp.pad(W2_l, ((0, P16 - OUT_CH), (0, 0))).T
    w2rT = jnp.pad(W2_r, ((0, P16 - OUT_CH), (0, 0))).T
    b2r = jnp.pad(b2, (0, P16 - OUT_CH)).reshape(1, P16)
    b1r = b1.reshape(1, HID_CH)

    agg2 = _sc_agg_wide(x2, srcA, dstA)
    p_pad, qb, invb = _tc_dense(agg2, x2, w1lT, w1rT, b1r, w2lT, w2rT, b2r)
    pparts = _sc_agg_small(p_pad, srcC, dstC)
    out16 = _tc_combine(pparts, invb, qb)
    return out16[:N_NODES, :OUT_CH]


# speedup vs baseline: 1.2174x; 1.1084x over previous
import functools

import jax
import jax.numpy as jnp
from jax import lax
from jax.experimental import pallas as pl
from jax.experimental.pallas import tpu as pltpu
from jax.experimental.pallas import tpu_sc as plsc

N_NODES = 10000
N_EDGES = 320000
IN_CH = 128
HID_CH = 256
OUT_CH = 2

W80 = 80
CNT_COL = 48
P16 = 16
NC = 2
NS = 16
N_PAD = 10240
SROWS = N_PAD // NS
KA = 80
TA = 250

KC = 80
EPW_C = N_EDGES // (NC * NS)
TC_CH = 126


def _zero_fill(buf, rows, cols):
    def zrow(r, _):
        def zcol(c, _):
            buf[r, pl.ds(c * 16, 16)] = jnp.zeros((16,), jnp.float32)
            return 0
        return lax.fori_loop(0, cols // 16, zcol, 0)
    lax.fori_loop(0, rows, zrow, 0)


def _zero_shared_slice(buf, shared, base, bufrows):
    for t in range(SROWS // bufrows):
        pltpu.sync_copy(buf, shared.at[pl.ds(base + t * bufrows, bufrows)])


def _edge_loop(table_hbm, src_v, dst_v, shared, b0, b1, g0, g1, nch):
    pltpu.async_copy(table_hbm.at[src_v.at[0]], b0, g0)
    pltpu.async_copy(table_hbm.at[src_v.at[1]], b1, g1)

    def step(j, buf, sem):
        pltpu.make_async_copy(table_hbm.at[src_v.at[j]], buf, sem).wait()
        pltpu.sync_copy(buf, shared.at[dst_v.at[j]], add=True)
        pltpu.async_copy(table_hbm.at[src_v.at[j + 2]], buf, sem)

    def body(i, _):
        step(2 * i, b0, g0)
        step(2 * i + 1, b1, g1)
        return 0

    lax.fori_loop(0, nch // 2, body, 0)
    pltpu.make_async_copy(table_hbm.at[src_v.at[0]], b0, g0).wait()
    pltpu.make_async_copy(table_hbm.at[src_v.at[0]], b1, g1).wait()


def _sc_agg_wide(x2, src3, dst3):
    mesh = plsc.VectorSubcoreMesh(core_axis_name="c", subcore_axis_name="s")

    @functools.partial(
        pl.kernel,
        out_type=jax.ShapeDtypeStruct((NC, N_PAD, W80), jnp.float32),
        mesh=mesh,
        scratch_types=[
            pltpu.VMEM((TA + 2, KA), jnp.int32),
            pltpu.VMEM((TA, KA), jnp.int32),
            pltpu.VMEM((KA, W80), jnp.float32),
            pltpu.VMEM((KA, W80), jnp.float32),
            pltpu.VMEM_SHARED((N_PAD, W80), jnp.float32),
            pltpu.SemaphoreType.DMA,
            pltpu.SemaphoreType.DMA,
        ],
        compiler_params=pltpu.CompilerParams(use_tc_tiling_on_sc=False),
    )
    def k(x_hbm, src_hbm, dst_hbm, out_hbm, src_v, dst_v, b0, b1,
          shared, g0, g1):
        cid = lax.axis_index("c")
        sid = lax.axis_index("s")
        base = pl.multiple_of(sid * SROWS, SROWS)

        _zero_fill(b0, KA, W80)
        _zero_shared_slice(b0, shared, base, KA)
        pltpu.sync_copy(src_hbm.at[sid], src_v)
        pltpu.sync_copy(dst_hbm.at[sid], dst_v)

        off = jnp.full((16,), cid * N_PAD, jnp.int32)

        def orow(r, _):
            def ocol(c, _):
                src_v[r, pl.ds(c * 16, 16)] = (
                    src_v[r, pl.ds(c * 16, 16)] + off)
                return 0
            return lax.fori_loop(0, KA // 16, ocol, 0)
        lax.fori_loop(0, TA + 2, orow, 0)

        plsc.subcore_barrier()
        _edge_loop(x_hbm, src_v, dst_v, shared, b0, b1, g0, g1, TA)
        plsc.subcore_barrier()

        pltpu.sync_copy(shared.at[pl.ds(base, SROWS)],
                        out_hbm.at[cid, pl.ds(base, SROWS)])

    return k(x2, src3, dst3)


def _sc_agg_small(p_pad, src3, dst3):
    mesh = plsc.VectorSubcoreMesh(core_axis_name="c", subcore_axis_name="s")

    @functools.partial(
        pl.kernel,
        out_type=jax.ShapeDtypeStruct((NC, N_PAD, P16), jnp.float32),
        mesh=mesh,
        scratch_types=[
            pltpu.VMEM((TC_CH + 2, KC), jnp.int32),
            pltpu.VMEM((TC_CH, KC), jnp.int32),
            pltpu.VMEM((KC, P16), jnp.float32),
            pltpu.VMEM((KC, P16), jnp.float32),
            pltpu.VMEM_SHARED((N_PAD, P16), jnp.float32),
            pltpu.SemaphoreType.DMA,
            pltpu.SemaphoreType.DMA,
        ],
        compiler_params=pltpu.CompilerParams(use_tc_tiling_on_sc=False),
    )
    def k(p_hbm, src_hbm, dst_hbm, out_hbm, src_v, dst_v, b0, b1,
          shared, g0, g1):
        cid = lax.axis_index("c")
        sid = lax.axis_index("s")
        wid = cid * NS + sid
        base = pl.multiple_of(sid * SROWS, SROWS)

        _zero_fill(b0, KC, P16)
        _zero_shared_slice(b0, shared, base, KC)
        pltpu.sync_copy(src_hbm.at[wid], src_v)
        pltpu.sync_copy(dst_hbm.at[wid], dst_v)
        plsc.subcore_barrier()
        _edge_loop(p_hbm, src_v, dst_v, shared, b0, b1, g0, g1, TC_CH)
        plsc.subcore_barrier()

        pltpu.sync_copy(shared.at[pl.ds(base, SROWS)],
                        out_hbm.at[cid, pl.ds(base, SROWS)])

    return k(p_pad, src3, dst3)


def _tc_dense(agg2, x2, w1lT, w1rT, b1r, w2lT, w2rT, b2r):
    NB = 1024
    grid = (N_PAD // NB,)

    def body(a_ref, xlo_ref, xhi_ref, w1l_ref, w1r_ref, b1_ref, w2l_ref,
             w2r_ref, b2_ref, pout_ref, qout_ref, inv_ref):
        alo = a_ref[0]
        ahi = a_ref[1]
        cnt = ahi[:, CNT_COL:CNT_COL + 1]
        inv = 1.0 / jnp.maximum(cnt, 1.0)
        mean = jnp.concatenate([alo, ahi[:, :IN_CH - W80]], axis=1) * inv
        x_blk = jnp.concatenate(
            [xlo_ref[...], xhi_ref[...][:, :IN_CH - W80]], axis=1)
        h = (jnp.dot(mean, w1l_ref[...], preferred_element_type=jnp.float32)
             + b1_ref[...]
             + jnp.dot(x_blk, w1r_ref[...],
                       preferred_element_type=jnp.float32))
        h = jnp.maximum(h, 0.0)
        pout_ref[...] = jnp.dot(h, w2l_ref[...],
                                preferred_element_type=jnp.float32)
        qout_ref[...] = (jnp.dot(h, w2r_ref[...],
                                 preferred_element_type=jnp.float32)
                         + b2_ref[...])
        inv_ref[...] = jnp.broadcast_to(inv, (NB, P16))

    return pl.pallas_call(
        body,
        grid=grid,
        in_specs=[
            pl.BlockSpec((NC, NB, W80), lambda i: (0, i, 0)),
            pl.BlockSpec((NB, W80), lambda i: (i, 0)),
            pl.BlockSpec((NB, W80), lambda i: (i + N_PAD // NB, 0)),
            pl.BlockSpec((IN_CH, HID_CH), lambda i: (0, 0)),
            pl.BlockSpec((IN_CH, HID_CH), lambda i: (0, 0)),
            pl.BlockSpec((1, HID_CH), lambda i: (0, 0)),
            pl.BlockSpec((HID_CH, P16), lambda i: (0, 0)),
            pl.BlockSpec((HID_CH, P16), lambda i: (0, 0)),
            pl.BlockSpec((1, P16), lambda i: (0, 0)),
        ],
        out_specs=[
            pl.BlockSpec((NB, P16), lambda i: (i, 0)),
            pl.BlockSpec((NB, P16), lambda i: (i, 0)),
            pl.BlockSpec((NB, P16), lambda i: (i, 0)),
        ],
        out_shape=[
            jax.ShapeDtypeStruct((N_PAD, P16), jnp.float32),
            jax.ShapeDtypeStruct((N_PAD, P16), jnp.float32),
            jax.ShapeDtypeStruct((N_PAD, P16), jnp.float32),
        ],
    )(agg2, x2, x2, w1lT, w1rT, b1r, w2lT, w2rT, b2r)


def _tc_combine(pparts, invb, qb):
    def body(p_ref, i_ref, q_ref, o_ref):
        o_ref[...] = ((p_ref[0] + p_ref[1]) * i_ref[...]) + q_ref[...]

    return pl.pallas_call(
        body,
        out_shape=jax.ShapeDtypeStruct((N_PAD, P16), jnp.float32),
    )(pparts, invb, qb)


def kernel(x, edge_index, W1_l, b1, W1_r, W2_l, b2, W2_r):
    ei = edge_index.astype(jnp.int32)
    src, dst = ei[0], ei[1]

    x_lo = jnp.pad(x[:, :W80], ((0, N_PAD - N_NODES), (0, 0)))
    x_hi = jnp.concatenate(
        [x[:, W80:], jnp.ones((N_NODES, 1), x.dtype),
         jnp.zeros((N_NODES, W80 - CNT_COL - 1), x.dtype)], axis=1)
    x_hi = jnp.pad(x_hi, ((0, N_PAD - N_NODES), (0, 0)))
    x2 = jnp.concatenate([x_lo, x_hi], axis=0)

    def pack(nw, t, k):
        epw = N_EDGES // nw
        pad_e = t * k - epw
        s3 = jnp.pad(src.reshape(nw, epw), ((0, 0), (0, pad_e)))
        s3 = jnp.pad(s3.reshape(nw, t, k), ((0, 0), (0, 2), (0, 0)))
        dummy = jnp.broadcast_to(
            (N_NODES + 100 + jnp.arange(nw, dtype=jnp.int32))[:, None],
            (nw, pad_e))
        d3 = jnp.concatenate(
            [dst.reshape(nw, epw), dummy], axis=1).reshape(nw, t, k)
        return s3, d3

    srcA, dstA = pack(NS, TA, KA)
    srcC, dstC = pack(NC * NS, TC_CH, KC)

    w1lT = W1_l.T
    w1rT = W1_r.T
    w2lT = jnp.pad(W2_l, ((0, P16 - OUT_CH), (0, 0))).T
    w2rT = jnp.pad(W2_r, ((0, P16 - OUT_CH), (0, 0))).T
    b2r = jnp.pad(b2, (0, P16 - OUT_CH)).reshape(1, P16)
    b1r = b1.reshape(1, HID_CH)

    agg2 = _sc_agg_wide(x2, srcA, dstA)
    p_pad, qb, invb = _tc_dense(agg2, x2, w1lT, w1rT, b1r, w2lT, w2rT, b2r)
    pparts = _sc_agg_small(p_pad, srcC, dstC)
    out16 = _tc_combine(pparts, invb, qb)
    return out16[:N_NODES, :OUT_CH]
